# baseline (speedup 1.0000x reference)
import jax
import jax.numpy as jnp
from jax.experimental import pallas as pl
from jax.experimental.pallas import tpu as pltpu

NQ = 4
NSUB = 2


def _moe_body(x_hbm, gw_ref, w_hbm, em_ref, ow_ref, ob_ref,
              gates_ref, y_ref, emo_ref, h_vmem, *rest):
    w_bufs, sems = rest[:NQ], rest[NQ]
    E = w_hbm.shape[0]
    ce = E // NQ

    hcp = pltpu.make_async_copy(x_hbm.at[:, 0, :], h_vmem, sems.at[NQ])
    hcp.start()
    wcopies = [
        pltpu.make_async_copy(
            w_hbm.at[pl.ds(q * ce, ce)], w_bufs[q], sems.at[q])
        for q in range(NQ)
    ]
    for c in wcopies:
        c.start()

    ow = ow_ref[...]
    b = ob_ref[0, 0]

    vmin = jax.lax.dot_general(
        em_ref[...], ow, (((1,), (1,)), ((), ())),
        preferred_element_type=jnp.float32)

    hcp.wait()
    h = h_vmem[...]

    gates_ref[...] = jax.lax.dot_general(
        h, gw_ref[...], (((1,), (1,)), ((), ())),
        preferred_element_type=jnp.float32)

    emo_ref[...] = jax.lax.dot_general(
        h, vmin, (((1,), (0,)), ((), ()))) + b

    cs = ce // NSUB
    vparts = []
    for q, c in enumerate(wcopies):
        c.wait()
        for j in range(NSUB):
            vparts.append(
                jnp.sum(w_bufs[q][pl.ds(j * cs, cs)] * ow[None, :, :],
                        axis=2))
    v = jnp.concatenate(vparts, axis=0)

    y_ref[...] = jax.lax.dot_general(
        h, v, (((1,), (1,)), ((), ())),
        preferred_element_type=jnp.float32) + b


def kernel(x, gate_weights, expert_weights, expert_min, out_w, out_b):
    B, _, H = x.shape
    E, _, S = expert_weights.shape
    ob2 = out_b.reshape(1, 1)

    gates, y2, emo = pl.pallas_call(
        _moe_body,
        in_specs=[
            pl.BlockSpec(memory_space=pltpu.MemorySpace.HBM),
            pl.BlockSpec(memory_space=pltpu.VMEM),
            pl.BlockSpec(memory_space=pltpu.MemorySpace.HBM),
            pl.BlockSpec(memory_space=pltpu.VMEM),
            pl.BlockSpec(memory_space=pltpu.VMEM),
            pl.BlockSpec(memory_space=pltpu.VMEM),
        ],
        out_shape=[
            jax.ShapeDtypeStruct((B, E), jnp.float32),
            jax.ShapeDtypeStruct((B, E), jnp.float32),
            jax.ShapeDtypeStruct((B, 1), jnp.float32),
        ],
        scratch_shapes=[pltpu.VMEM((B, H), jnp.float32)]
        + [pltpu.VMEM((E // NQ, H, S), jnp.float32) for _ in range(NQ)]
        + [pltpu.SemaphoreType.DMA((NQ + 1,))],
    )(x, gate_weights, expert_weights, expert_min, out_w, ob2)

    return (gates, y2.reshape(B, E, 1), emo)
